# Initial kernel scaffold; baseline (speedup 1.0000x reference)
import jax
import jax.numpy as jnp
from jax.experimental import pallas as pl


def kernel(attn, relative_position_bias_table, relative_position_index):
    raise NotImplementedError("write your pallas kernel here")


# speedup vs baseline: 6.2695x; 6.2695x over previous
import functools

import jax
import jax.numpy as jnp
from jax import lax
from jax.experimental import pallas as pl
from jax.experimental.pallas import tpu as pltpu
from jax.experimental.pallas import tpu_sc as plsc

WIN_AREA = 1024
NPOS = WIN_AREA * WIN_AREA
NHEADS = 16
NDIST = 3969

NC, NS, L = 2, 16, 16
NW = NC * NS
POS_PER_W = NPOS // NW
CHUNK = 2048
N_CHUNKS = POS_PER_W // CHUNK


def _sc_bias_kernel(table_hbm, idx_hbm, bias_hbm, table_v, idx_v, buf_v, sem):
    wid = lax.axis_index("s") * NC + lax.axis_index("c")
    base = wid * POS_PER_W

    pltpu.sync_copy(table_hbm, table_v)

    def chunk_body(c, _):
        pos0 = base + c * CHUNK
        pltpu.sync_copy(idx_hbm.at[pl.ds(pos0, CHUNK)], idx_v)

        def group_body(k, _):
            iv = idx_v[pl.ds(k * L, L)] * NHEADS
            for h in range(NHEADS):
                vals = plsc.load_gather(table_v, [iv + h])
                buf_v[h, pl.ds(k * L, L)] = vals
            return ()

        lax.fori_loop(0, CHUNK // L, group_body, (), unroll=False)
        pltpu.sync_copy(buf_v, bias_hbm.at[:, pl.ds(pos0, CHUNK)])
        return ()

    lax.fori_loop(0, N_CHUNKS, chunk_body, (), unroll=False)


def _sc_build_bias(table, idx):
    mesh = plsc.VectorSubcoreMesh(core_axis_name="c", subcore_axis_name="s")
    return pl.kernel(
        _sc_bias_kernel,
        out_type=jax.ShapeDtypeStruct((NHEADS, NPOS), jnp.float32),
        mesh=mesh,
        compiler_params=pltpu.CompilerParams(needs_layout_passes=False),
        scratch_types=[
            pltpu.VMEM((NDIST * NHEADS,), jnp.float32),
            pltpu.VMEM((CHUNK,), jnp.int32),
            pltpu.VMEM((NHEADS, CHUNK), jnp.float32),
            pltpu.SemaphoreType.DMA,
        ],
    )(table, idx)


BI = 64


def _tc_add_kernel(attn_ref, bias_ref, out_ref):
    out_ref[...] = attn_ref[...] + bias_ref[...][None]


def _tc_add(attn, bias3):
    grid = (WIN_AREA // BI, attn.shape[0])
    return pl.pallas_call(
        _tc_add_kernel,
        grid=grid,
        in_specs=[
            pl.BlockSpec((1, NHEADS, BI, WIN_AREA), lambda ib, b: (b, 0, ib, 0)),
            pl.BlockSpec((NHEADS, BI, WIN_AREA), lambda ib, b: (0, ib, 0)),
        ],
        out_specs=pl.BlockSpec((1, NHEADS, BI, WIN_AREA), lambda ib, b: (b, 0, ib, 0)),
        out_shape=jax.ShapeDtypeStruct(attn.shape, attn.dtype),
    )(attn, bias3)


@jax.jit
def kernel(attn, relative_position_bias_table, relative_position_index):
    bias_t = _sc_build_bias(relative_position_bias_table.reshape(-1),
                            relative_position_index)
    bias3 = bias_t.reshape(NHEADS, WIN_AREA, WIN_AREA)
    return _tc_add(attn, bias3)


# speedup vs baseline: 6.4663x; 1.0314x over previous
import functools

import jax
import jax.numpy as jnp
from jax import lax
from jax.experimental import pallas as pl
from jax.experimental.pallas import tpu as pltpu
from jax.experimental.pallas import tpu_sc as plsc

WIN_AREA = 1024
NPOS = WIN_AREA * WIN_AREA
NHEADS = 16
NDIST = 3969

NC, NS, L = 2, 16, 16
NW = NC * NS
POS_PER_W = NPOS // NW
CHUNK = 2048
N_CHUNKS = POS_PER_W // CHUNK


def _sc_bias_kernel(table_hbm, idx_hbm, bias_hbm, table_v, idx_v, buf_v, sem):
    wid = lax.axis_index("s") * NC + lax.axis_index("c")
    base = wid * POS_PER_W

    pltpu.sync_copy(table_hbm, table_v)

    def chunk_body(c, _):
        pos0 = base + c * CHUNK
        pltpu.sync_copy(idx_hbm.at[pl.ds(pos0, CHUNK)], idx_v)

        def group_body(k, _):
            iv = idx_v[pl.ds(k * L, L)] * NHEADS
            for h in range(NHEADS):
                vals = plsc.load_gather(table_v, [iv + h])
                buf_v[h, pl.ds(k * L, L)] = vals
            return ()

        lax.fori_loop(0, CHUNK // L, group_body, (), unroll=False)
        pltpu.sync_copy(buf_v, bias_hbm.at[:, pl.ds(pos0, CHUNK)])
        return ()

    lax.fori_loop(0, N_CHUNKS, chunk_body, (), unroll=False)


def _sc_build_bias(table, idx):
    mesh = plsc.VectorSubcoreMesh(core_axis_name="c", subcore_axis_name="s")
    return pl.kernel(
        _sc_bias_kernel,
        out_type=jax.ShapeDtypeStruct((NHEADS, NPOS), jnp.float32),
        mesh=mesh,
        compiler_params=pltpu.CompilerParams(needs_layout_passes=False),
        scratch_types=[
            pltpu.VMEM((NDIST * NHEADS,), jnp.float32),
            pltpu.VMEM((CHUNK,), jnp.int32),
            pltpu.VMEM((NHEADS, CHUNK), jnp.float32),
            pltpu.SemaphoreType.DMA,
        ],
    )(table, idx)


BI = 16


def _tc_add_kernel(attn_ref, bias_ref, out_ref):
    out_ref[...] = attn_ref[...] + bias_ref[...][None]


def _tc_add(attn, bias3):
    nb = attn.shape[0]
    return pl.pallas_call(
        _tc_add_kernel,
        grid=(WIN_AREA // BI,),
        in_specs=[
            pl.BlockSpec((nb, NHEADS, BI, WIN_AREA), lambda ib: (0, 0, ib, 0)),
            pl.BlockSpec((NHEADS, BI, WIN_AREA), lambda ib: (0, ib, 0)),
        ],
        out_specs=pl.BlockSpec((nb, NHEADS, BI, WIN_AREA), lambda ib: (0, 0, ib, 0)),
        out_shape=jax.ShapeDtypeStruct(attn.shape, attn.dtype),
    )(attn, bias3)


@jax.jit
def kernel(attn, relative_position_bias_table, relative_position_index):
    bias_t = _sc_build_bias(relative_position_bias_table.reshape(-1),
                            relative_position_index)
    bias3 = bias_t.reshape(NHEADS, WIN_AREA, WIN_AREA)
    return _tc_add(attn, bias3)


# speedup vs baseline: 8.0338x; 1.2424x over previous
import functools

import jax
import jax.numpy as jnp
from jax import lax
from jax.experimental import pallas as pl
from jax.experimental.pallas import tpu as pltpu
from jax.experimental.pallas import tpu_sc as plsc

WIN_AREA = 1024
NPOS = WIN_AREA * WIN_AREA
NHEADS = 16
NDIST = 3969

NC, NS, L = 2, 16, 16
NW = NC * NS
POS_PER_W = NPOS // NW
CHUNK = 1024
N_CHUNKS = POS_PER_W // CHUNK
UNROLL = 4


def _sc_bias_kernel(table_hbm, idx_hbm, bias_hbm, table_v, idx_v, buf_v, sem):
    wid = lax.axis_index("s") * NC + lax.axis_index("c")
    base = wid * POS_PER_W

    pltpu.sync_copy(table_hbm, table_v)

    def chunk_body(c, _):
        pos0 = base + c * CHUNK
        pltpu.sync_copy(idx_hbm.at[pl.ds(pos0, CHUNK)], idx_v)

        @plsc.parallel_loop(0, CHUNK // L, unroll=UNROLL)
        def group_body(k):
            iv = idx_v[pl.ds(k * L, L)] * NHEADS
            for h in range(NHEADS):
                buf_v[h, pl.ds(k * L, L)] = plsc.load_gather(
                    table_v, [iv + h])

        pltpu.sync_copy(buf_v, bias_hbm.at[:, pl.ds(pos0, CHUNK)])
        return ()

    lax.fori_loop(0, N_CHUNKS, chunk_body, (), unroll=False)


def _sc_build_bias(table, idx):
    mesh = plsc.VectorSubcoreMesh(core_axis_name="c", subcore_axis_name="s")
    return pl.kernel(
        _sc_bias_kernel,
        out_type=jax.ShapeDtypeStruct((NHEADS, NPOS), jnp.float32),
        mesh=mesh,
        compiler_params=pltpu.CompilerParams(needs_layout_passes=False),
        scratch_types=[
            pltpu.VMEM((NDIST * NHEADS,), jnp.float32),
            pltpu.VMEM((CHUNK,), jnp.int32),
            pltpu.VMEM((NHEADS, CHUNK), jnp.float32),
            pltpu.SemaphoreType.DMA,
        ],
    )(table, idx)


BI = 16


def _tc_add_kernel(attn_ref, bias_ref, out_ref):
    out_ref[...] = attn_ref[...] + bias_ref[...][None]


def _tc_add(attn, bias3):
    nb = attn.shape[0]
    return pl.pallas_call(
        _tc_add_kernel,
        grid=(WIN_AREA // BI,),
        in_specs=[
            pl.BlockSpec((nb, NHEADS, BI, WIN_AREA), lambda ib: (0, 0, ib, 0)),
            pl.BlockSpec((NHEADS, BI, WIN_AREA), lambda ib: (0, ib, 0)),
        ],
        out_specs=pl.BlockSpec((nb, NHEADS, BI, WIN_AREA), lambda ib: (0, 0, ib, 0)),
        out_shape=jax.ShapeDtypeStruct(attn.shape, attn.dtype),
    )(attn, bias3)


@jax.jit
def kernel(attn, relative_position_bias_table, relative_position_index):
    bias_t = _sc_build_bias(relative_position_bias_table.reshape(-1),
                            relative_position_index)
    bias3 = bias_t.reshape(NHEADS, WIN_AREA, WIN_AREA)
    return _tc_add(attn, bias3)


# speedup vs baseline: 8.7347x; 1.0872x over previous
import functools

import jax
import jax.numpy as jnp
from jax import lax
from jax.experimental import pallas as pl
from jax.experimental.pallas import tpu as pltpu
from jax.experimental.pallas import tpu_sc as plsc

WIN_AREA = 1024
NPOS = WIN_AREA * WIN_AREA
NHEADS = 16
NDIST = 3969

NC, NS, L = 2, 16, 16
NW = NC * NS
POS_PER_W = NPOS // NW
CHUNK = 1024
N_CHUNKS = POS_PER_W // CHUNK
UNROLL = 4


def _sc_bias_kernel(table_hbm, idx_hbm, bias_hbm, table_v, idx_v, buf_v, sem):
    wid = lax.axis_index("s") * NC + lax.axis_index("c")
    base = wid * POS_PER_W

    pltpu.sync_copy(table_hbm, table_v)

    def chunk_body(c, _):
        pos0 = base + c * CHUNK
        pltpu.sync_copy(idx_hbm.at[pl.ds(pos0, CHUNK)], idx_v)

        @plsc.parallel_loop(0, CHUNK // L, unroll=UNROLL)
        def group_body(k):
            iv = idx_v[pl.ds(k * L, L)] * NHEADS
            for h in range(NHEADS):
                buf_v[h, pl.ds(k * L, L)] = plsc.load_gather(
                    table_v, [iv + h])

        pltpu.sync_copy(buf_v, bias_hbm.at[:, wid * N_CHUNKS + c])
        return ()

    lax.fori_loop(0, N_CHUNKS, chunk_body, (), unroll=False)


def _sc_build_bias(table, idx):
    mesh = plsc.VectorSubcoreMesh(core_axis_name="c", subcore_axis_name="s")
    return pl.kernel(
        _sc_bias_kernel,
        out_type=jax.ShapeDtypeStruct((NHEADS, WIN_AREA, WIN_AREA), jnp.float32),
        mesh=mesh,
        compiler_params=pltpu.CompilerParams(needs_layout_passes=False),
        scratch_types=[
            pltpu.VMEM((NDIST * NHEADS,), jnp.float32),
            pltpu.VMEM((CHUNK,), jnp.int32),
            pltpu.VMEM((NHEADS, CHUNK), jnp.float32),
            pltpu.SemaphoreType.DMA,
        ],
    )(table, idx)


BI = 16


def _tc_add_kernel(attn_ref, bias_ref, out_ref):
    out_ref[...] = attn_ref[...] + bias_ref[...][None]


def _tc_add(attn, bias3):
    nb = attn.shape[0]
    return pl.pallas_call(
        _tc_add_kernel,
        grid=(WIN_AREA // BI,),
        in_specs=[
            pl.BlockSpec((nb, NHEADS, BI, WIN_AREA), lambda ib: (0, 0, ib, 0)),
            pl.BlockSpec((NHEADS, BI, WIN_AREA), lambda ib: (0, ib, 0)),
        ],
        out_specs=pl.BlockSpec((nb, NHEADS, BI, WIN_AREA), lambda ib: (0, 0, ib, 0)),
        out_shape=jax.ShapeDtypeStruct(attn.shape, attn.dtype),
    )(attn, bias3)


@jax.jit
def kernel(attn, relative_position_bias_table, relative_position_index):
    bias3 = _sc_build_bias(relative_position_bias_table.reshape(-1),
                           relative_position_index)
    return _tc_add(attn, bias3)


# speedup vs baseline: 9.7517x; 1.1164x over previous
import functools

import jax
import jax.numpy as jnp
from jax import lax
from jax.experimental import pallas as pl
from jax.experimental.pallas import tpu as pltpu
from jax.experimental.pallas import tpu_sc as plsc

WIN_AREA = 1024
NPOS = WIN_AREA * WIN_AREA
NHEADS = 16
NDIST = 3969

NC, NS, L = 2, 16, 16
NW = NC * NS
POS_PER_W = NPOS // NW
CHUNK = 1024
N_CHUNKS = POS_PER_W // CHUNK
UNROLL = 4


def _sc_bias_kernel(table_hbm, idx_hbm, bias_hbm, table_v, idx_v,
                    buf_a, buf_b, sem_t, sem_i, sem_a, sem_b):
    wid = lax.axis_index("s") * NC + lax.axis_index("c")
    base = wid * POS_PER_W
    row0 = wid * N_CHUNKS

    tcopy = pltpu.async_copy(table_hbm, table_v, sem_t)
    icopy = pltpu.async_copy(idx_hbm.at[pl.ds(base, POS_PER_W)], idx_v, sem_i)
    tcopy.wait()
    icopy.wait()

    def gather_chunk(c, buf):
        @plsc.parallel_loop(0, CHUNK // L, unroll=UNROLL)
        def group_body(k):
            iv = idx_v[pl.ds(c * CHUNK + k * L, L)] * NHEADS
            for h in range(NHEADS):
                buf[h, pl.ds(k * L, L)] = plsc.load_gather(table_v, [iv + h])

    def put_chunk(c, buf, sem):
        pltpu.async_copy(buf, bias_hbm.at[:, row0 + c], sem)

    def wait_chunk(buf, sem):
        pltpu.make_async_copy(buf, bias_hbm.at[:, row0], sem).wait()

    gather_chunk(0, buf_a)
    put_chunk(0, buf_a, sem_a)
    gather_chunk(1, buf_b)
    put_chunk(1, buf_b, sem_b)

    def pair_body(p, _):
        c = p * 2
        wait_chunk(buf_a, sem_a)
        gather_chunk(c, buf_a)
        put_chunk(c, buf_a, sem_a)
        wait_chunk(buf_b, sem_b)
        gather_chunk(c + 1, buf_b)
        put_chunk(c + 1, buf_b, sem_b)
        return ()

    lax.fori_loop(1, N_CHUNKS // 2, pair_body, (), unroll=False)
    wait_chunk(buf_a, sem_a)
    wait_chunk(buf_b, sem_b)


def _sc_build_bias(table, idx):
    mesh = plsc.VectorSubcoreMesh(core_axis_name="c", subcore_axis_name="s")
    return pl.kernel(
        _sc_bias_kernel,
        out_type=jax.ShapeDtypeStruct((NHEADS, WIN_AREA, WIN_AREA), jnp.float32),
        mesh=mesh,
        compiler_params=pltpu.CompilerParams(needs_layout_passes=False),
        scratch_types=[
            pltpu.VMEM((NDIST * NHEADS,), jnp.float32),
            pltpu.VMEM((POS_PER_W,), jnp.int32),
            pltpu.VMEM((NHEADS, CHUNK), jnp.float32),
            pltpu.VMEM((NHEADS, CHUNK), jnp.float32),
            pltpu.SemaphoreType.DMA,
            pltpu.SemaphoreType.DMA,
            pltpu.SemaphoreType.DMA,
            pltpu.SemaphoreType.DMA,
        ],
    )(table, idx)


BI = 16


def _tc_add_kernel(attn_ref, bias_ref, out_ref):
    out_ref[...] = attn_ref[...] + bias_ref[...][None]


def _tc_add(attn, bias3):
    nb = attn.shape[0]
    return pl.pallas_call(
        _tc_add_kernel,
        grid=(WIN_AREA // BI,),
        in_specs=[
            pl.BlockSpec((nb, NHEADS, BI, WIN_AREA), lambda ib: (0, 0, ib, 0)),
            pl.BlockSpec((NHEADS, BI, WIN_AREA), lambda ib: (0, ib, 0)),
        ],
        out_specs=pl.BlockSpec((nb, NHEADS, BI, WIN_AREA), lambda ib: (0, 0, ib, 0)),
        out_shape=jax.ShapeDtypeStruct(attn.shape, attn.dtype),
    )(attn, bias3)


@jax.jit
def kernel(attn, relative_position_bias_table, relative_position_index):
    bias3 = _sc_build_bias(relative_position_bias_table.reshape(-1),
                           relative_position_index)
    return _tc_add(attn, bias3)
